# Initial kernel scaffold; baseline (speedup 1.0000x reference)
import jax
import jax.numpy as jnp
from jax.experimental import pallas as pl


def kernel(node_types, type_embedding):
    raise NotImplementedError("write your pallas kernel here")


# speedup vs baseline: 2.9579x; 2.9579x over previous
import functools

import jax
import jax.numpy as jnp
from jax import lax
from jax.experimental import pallas as pl
from jax.experimental.pallas import tpu as pltpu
from jax.experimental.pallas import tpu_sc as plsc

_NUM_WORKERS = 32


def _plan(num_rows):
    assert num_rows % 8 == 0, num_rows
    granules = num_rows // 8
    lo = granules // _NUM_WORKERS
    nbig = granules - lo * _NUM_WORKERS
    small = lo * 8
    big = small + 8
    chunk = 8
    for c in range(min(small, 960), 0, -8):
        if small % c == 0:
            chunk = c
            break
    return big, small, nbig, chunk, small // chunk


@functools.lru_cache(maxsize=None)
def _make(num_rows, dim):
    big, small, nbig, chunk, nchunks = _plan(num_rows)
    mesh = plsc.VectorSubcoreMesh(core_axis_name="c", subcore_axis_name="s")

    @functools.partial(
        pl.kernel,
        mesh=mesh,
        out_type=jax.ShapeDtypeStruct((num_rows, dim), jnp.float32),
        scratch_types=[
            pltpu.VMEM((chunk,), jnp.int32),
            pltpu.VMEM((chunk, dim), jnp.float32),
            pltpu.VMEM((8,), jnp.int32),
            pltpu.VMEM((8, dim), jnp.float32),
            pltpu.SemaphoreType.DMA,
        ],
    )
    def gather_kernel(idx_hbm, table_hbm, out_hbm, idx_v, rows_v, idx_t,
                      rows_t, sem):
        wid = lax.axis_index("s") * 2 + lax.axis_index("c")
        base = jnp.where(wid < nbig, wid * big,
                         nbig * big + (wid - nbig) * small)
        base = pl.multiple_of(base, 8)

        def body(j, carry):
            start = pl.multiple_of(base + j * chunk, 8)
            pltpu.sync_copy(idx_hbm.at[pl.ds(start, chunk)], idx_v)
            pltpu.async_copy(table_hbm.at[idx_v], rows_v, sem).wait()
            pltpu.sync_copy(rows_v, out_hbm.at[pl.ds(start, chunk), :])
            return carry

        lax.fori_loop(0, nchunks, body, 0)

        if nbig:
            @pl.when(wid < nbig)
            def _tail():
                start = pl.multiple_of(base + nchunks * chunk, 8)
                pltpu.sync_copy(idx_hbm.at[pl.ds(start, 8)], idx_t)
                pltpu.async_copy(table_hbm.at[idx_t], rows_t, sem).wait()
                pltpu.sync_copy(rows_t, out_hbm.at[pl.ds(start, 8), :])

    return gather_kernel


def kernel(node_types, type_embedding):
    (num_rows,) = node_types.shape
    _, dim = type_embedding.shape
    idx = node_types.astype(jnp.int32)
    table = type_embedding.astype(jnp.float32)
    return _make(num_rows, dim)(idx, table)
